# baseline (speedup 1.0000x reference)
import jax
import jax.numpy as jnp
from jax import lax
from jax.experimental import pallas as pl
from jax.experimental.pallas import tpu as pltpu
from jax.experimental.pallas import tpu_sc as plsc

N = 50000
C = 64
E = 800000
HALF = 25000
HPAD = 25088
RPT = 1568
XROWS = 2 * HPAD
DUMP = 25080
BLK = 256
CHUNK = 2048
NBLK = CHUNK // BLK
NGRP = 25
EPT = CHUNK * NGRP
EPAD = 16 * EPT


NBUF = 4
GD = 3


def _sc_body(x_hbm, src_hbm, tgt_hbm, s1_hbm, s2_hbm,
             gidx, sidx, lidx, rows, accum, isem0, isem1, gsem, ssem):
    c = lax.axis_index("c")
    s = lax.axis_index("s")
    base = c * HALF

    for g_hbm, sc_hbm, out_hbm in ((src_hbm, tgt_hbm, s1_hbm),
                                   (tgt_hbm, src_hbm, s2_hbm)):
        pltpu.sync_copy(x_hbm.at[pl.ds(base + s * RPT, RPT)],
                        accum.at[pl.ds(s * RPT, RPT)])
        plsc.subcore_barrier()

        def group(g, carry):
            off = s * EPT + g * CHUNK
            d1 = pltpu.async_copy(g_hbm.at[pl.ds(off, CHUNK)], gidx, isem0)
            d2 = pltpu.async_copy(sc_hbm.at[pl.ds(off, CHUNK)], sidx, isem1)
            d1.wait()
            d2.wait()
            for b in range(NBLK):
                for j in range(BLK // 16):
                    v = sidx[pl.ds(b * BLK + j * 16, 16)]
                    lv = v - base
                    ok = (lv >= 0) & (lv < HALF)
                    lidx[b, pl.ds(j * 16, 16)] = jnp.where(ok, lv, DUMP)
            gd = [None] * NBLK
            sd = [None] * NBLK
            sdone = [False] * NBLK
            for b in range(min(GD, NBLK)):
                gd[b] = pltpu.async_copy(
                    x_hbm.at[gidx.at[pl.ds(b * BLK, BLK)]],
                    rows.at[b % NBUF], gsem[b % NBUF])
            for b in range(NBLK):
                gd[b].wait()
                sd[b] = pltpu.async_copy(
                    rows.at[b % NBUF], accum.at[lidx.at[b]],
                    ssem[b % NBUF], add=True)
                nb = b + GD
                if nb < NBLK:
                    prev = nb - NBUF
                    if prev >= 0:
                        sd[prev].wait()
                        sdone[prev] = True
                    gd[nb] = pltpu.async_copy(
                        x_hbm.at[gidx.at[pl.ds(nb * BLK, BLK)]],
                        rows.at[nb % NBUF], gsem[nb % NBUF])
            for b in range(NBLK):
                if not sdone[b]:
                    sd[b].wait()
            return carry

        lax.fori_loop(0, NGRP, group, 0)
        plsc.subcore_barrier()
        pltpu.sync_copy(accum.at[pl.ds(s * RPT, RPT)],
                        out_hbm.at[pl.ds(c * HPAD + s * RPT, RPT)])
        plsc.subcore_barrier()


def _tc_body(s1_ref, s2_ref, n_ref, nt_ref, w_ref, o_ref):
    a1 = s1_ref[0].astype(jnp.float32) * n_ref[...]
    a2 = s2_ref[0].astype(jnp.float32) * nt_ref[...]
    a = jnp.concatenate([a1, a2], axis=1)
    o_ref[...] = jnp.dot(a, w_ref[...], preferred_element_type=jnp.float32)


def kernel(x, sources, targets, norm, norm_t, W_out, W_back):
    src = jnp.asarray(sources, jnp.int32)
    tgt = jnp.asarray(targets, jnp.int32)
    pad = jnp.full((EPAD - E,), N, jnp.int32)
    srcp = jnp.concatenate([src, pad])
    tgtp = jnp.concatenate([tgt, pad])
    x_pad = jnp.zeros((XROWS, C), jnp.bfloat16).at[:N].set(x.astype(jnp.bfloat16))

    mesh = plsc.VectorSubcoreMesh(core_axis_name="c", subcore_axis_name="s")
    s1, s2 = pl.kernel(
        _sc_body,
        out_type=(jax.ShapeDtypeStruct((2 * HPAD, C), jnp.bfloat16),
                  jax.ShapeDtypeStruct((2 * HPAD, C), jnp.bfloat16)),
        mesh=mesh,
        scratch_types=[
            pltpu.VMEM((CHUNK,), jnp.int32),
            pltpu.VMEM((CHUNK,), jnp.int32),
            pltpu.VMEM((NBLK, BLK), jnp.int32),
            pltpu.VMEM((NBUF, BLK, C), jnp.bfloat16),
            pltpu.VMEM_SHARED((HPAD, C), jnp.bfloat16),
            pltpu.SemaphoreType.DMA,
            pltpu.SemaphoreType.DMA,
            [pltpu.SemaphoreType.DMA] * NBUF,
            [pltpu.SemaphoreType.DMA] * NBUF,
        ],
        compiler_params=pltpu.CompilerParams(use_tc_tiling_on_sc=False),
    )(x_pad, srcp, tgtp)

    s1_3 = s1.reshape(2, HPAD, C)
    s2_3 = s2.reshape(2, HPAD, C)
    W_cat = jnp.concatenate([W_out, W_back], axis=0)

    out = pl.pallas_call(
        _tc_body,
        grid=(2, 25),
        in_specs=[
            pl.BlockSpec((1, 1000, C), lambda h, i: (h, i, 0)),
            pl.BlockSpec((1, 1000, C), lambda h, i: (h, i, 0)),
            pl.BlockSpec((1000, 1), lambda h, i: (h * 25 + i, 0)),
            pl.BlockSpec((1000, 1), lambda h, i: (h * 25 + i, 0)),
            pl.BlockSpec((2 * C, C), lambda h, i: (0, 0)),
        ],
        out_specs=pl.BlockSpec((1000, C), lambda h, i: (h * 25 + i, 0)),
        out_shape=jax.ShapeDtypeStruct((N, C), jnp.float32),
    )(s1_3, s2_3, norm, norm_t, W_cat)
    return out


# speedup vs baseline: 2.7870x; 2.7870x over previous
import jax
import jax.numpy as jnp
from jax import lax
from jax.experimental import pallas as pl
from jax.experimental.pallas import tpu as pltpu
from jax.experimental.pallas import tpu_sc as plsc

N = 50000
C = 64
E = 800000
XROWS = 50176
RPT = 3136
BLK = 128
NBLK = 16
CHUNK = NBLK * BLK
NGRP = 13
EPT = CHUNK * NGRP
EPAD = 32 * EPT
NBUF = 6
GD = 5


def _sc_body(x2_hbm, src_hbm, tgt_hbm, s1_hbm, s2_hbm,
             gidx, sidx, rows, isem0, isem1, gsem, ssem, accum):
    c = lax.axis_index("c")
    s = lax.axis_index("s")
    tile = c * 16 + s

    for g_hbm, sc_hbm, out_hbm in ((src_hbm, tgt_hbm, s1_hbm),
                                   (tgt_hbm, src_hbm, s2_hbm)):
        pltpu.sync_copy(x2_hbm.at[pl.ds(c * XROWS + s * RPT, RPT)],
                        accum.at[pl.ds(s * RPT, RPT)])
        plsc.subcore_barrier()

        def group(g, carry):
            row0 = tile * (EPT // BLK) + g * NBLK
            d1 = pltpu.async_copy(g_hbm.at[pl.ds(row0, NBLK)], gidx, isem0)
            d2 = pltpu.async_copy(sc_hbm.at[pl.ds(row0, NBLK)], sidx, isem1)
            d1.wait()
            d2.wait()
            gd = [None] * NBLK
            sd = [None] * NBLK
            sdone = [False] * NBLK
            for b in range(min(GD, NBLK)):
                gd[b] = pltpu.async_copy(
                    x2_hbm.at[gidx.at[b]], rows.at[b % NBUF], gsem[b % NBUF])
            for b in range(NBLK):
                gd[b].wait()
                sd[b] = pltpu.async_copy(
                    rows.at[b % NBUF], accum.at[sidx.at[b]],
                    ssem[b % NBUF], add=True)
                nb = b + GD
                if nb < NBLK:
                    prev = nb - NBUF
                    if prev >= 0:
                        sd[prev].wait()
                        sdone[prev] = True
                    gd[nb] = pltpu.async_copy(
                        x2_hbm.at[gidx.at[nb]],
                        rows.at[nb % NBUF], gsem[nb % NBUF])
            for b in range(NBLK):
                if not sdone[b]:
                    sd[b].wait()
            return carry

        lax.fori_loop(0, NGRP, group, 0)
        plsc.subcore_barrier()
        pltpu.sync_copy(accum.at[pl.ds(s * RPT, RPT)],
                        out_hbm.at[pl.ds(c * XROWS + s * RPT, RPT)])
        plsc.subcore_barrier()


def _tc_body(s1_ref, s2_ref, n_ref, nt_ref, w_ref, o_ref):
    a1 = (s1_ref[0].astype(jnp.float32)
          + s1_ref[1].astype(jnp.float32)) * n_ref[...]
    a2 = (s2_ref[0].astype(jnp.float32)
          + s2_ref[1].astype(jnp.float32)) * nt_ref[...]
    a = jnp.concatenate([a1, a2], axis=1)
    o_ref[...] = jnp.dot(a, w_ref[...], preferred_element_type=jnp.float32)


def kernel(x, sources, targets, norm, norm_t, W_out, W_back):
    src = jnp.asarray(sources, jnp.int32)
    tgt = jnp.asarray(targets, jnp.int32)
    padv = N + (jnp.arange(EPAD - E, dtype=jnp.int32) % (XROWS - 8 - N))
    srcp = jnp.concatenate([src, padv]).reshape(EPAD // BLK, BLK)
    tgtp = jnp.concatenate([tgt, padv]).reshape(EPAD // BLK, BLK)
    x2 = jnp.zeros((2 * XROWS, C), jnp.bfloat16).at[:N].set(
        x.astype(jnp.bfloat16))

    mesh = plsc.VectorSubcoreMesh(core_axis_name="c", subcore_axis_name="s")
    s1, s2 = pl.kernel(
        _sc_body,
        out_type=(jax.ShapeDtypeStruct((2 * XROWS, C), jnp.bfloat16),
                  jax.ShapeDtypeStruct((2 * XROWS, C), jnp.bfloat16)),
        mesh=mesh,
        scratch_types=[
            pltpu.VMEM((NBLK, BLK), jnp.int32),
            pltpu.VMEM((NBLK, BLK), jnp.int32),
            pltpu.VMEM((NBUF, BLK, C), jnp.bfloat16),
            pltpu.SemaphoreType.DMA,
            pltpu.SemaphoreType.DMA,
            [pltpu.SemaphoreType.DMA] * NBUF,
            [pltpu.SemaphoreType.DMA] * NBUF,
            pltpu.VMEM_SHARED((XROWS, C), jnp.bfloat16),
        ],
        compiler_params=pltpu.CompilerParams(use_tc_tiling_on_sc=False),
    )(x2, srcp, tgtp)

    s1_3 = s1.reshape(2, XROWS, C)
    s2_3 = s2.reshape(2, XROWS, C)
    W_cat = jnp.concatenate([W_out, W_back], axis=0)

    out = pl.pallas_call(
        _tc_body,
        grid=(50,),
        in_specs=[
            pl.BlockSpec((2, 1000, C), lambda i: (0, i, 0)),
            pl.BlockSpec((2, 1000, C), lambda i: (0, i, 0)),
            pl.BlockSpec((1000, 1), lambda i: (i, 0)),
            pl.BlockSpec((1000, 1), lambda i: (i, 0)),
            pl.BlockSpec((2 * C, C), lambda i: (0, 0)),
        ],
        out_specs=pl.BlockSpec((1000, C), lambda i: (i, 0)),
        out_shape=jax.ShapeDtypeStruct((N, C), jnp.float32),
    )(s1_3, s2_3, norm, norm_t, W_cat)
    return out
